# Initial kernel scaffold; baseline (speedup 1.0000x reference)
import functools

import jax
import jax.numpy as jnp
from jax import lax
from jax.experimental import pallas as pl
from jax.experimental.pallas import tpu as pltpu
from jax.experimental.pallas import tpu_sc as plsc

N = 10000
E = 320000
F_IN = 128
H = 64
G = 64
EPS = 1e-5

N_PAD = 10240
CHUNK = 128
NC, NS = 2, 16
NW = NC * NS
CH_PER_TILE = 80
E_PAD = NW * CH_PER_TILE * CHUNK
ROWS_PER_TILE = N_PAD // NS
KDEPTH = 4

_mesh = plsc.VectorSubcoreMesh(core_axis_name="c", subcore_axis_name="s")


@functools.partial(
    pl.kernel,
    out_type=jax.ShapeDtypeStruct((NW, N_PAD), jnp.float32),
    mesh=_mesh,
    scratch_types=[
        pltpu.VMEM((CH_PER_TILE, CHUNK), jnp.int32),
        pltpu.VMEM((N_PAD,), jnp.float32),
    ],
)
def _sc_degree(dst_hbm, out_hbm, idx_v, deg_v):
    c = lax.axis_index("c")
    s = lax.axis_index("s")
    w = c * NS + s
    pltpu.sync_copy(dst_hbm.at[pl.ds(w * CH_PER_TILE, CH_PER_TILE)], idx_v)

    def zbody(i, carry):
        deg_v[pl.ds(i * 16, 16)] = jnp.zeros((16,), jnp.float32)
        return carry

    lax.fori_loop(0, N_PAD // 16, zbody, 0)

    ones = jnp.full((16,), 1.0, jnp.float32)

    def ebody(j, carry):
        for k in range(CHUNK // 16):
            idx16 = idx_v[j, pl.ds(k * 16, 16)]
            plsc.addupdate_scatter(deg_v, [idx16], ones)
        return carry

    lax.fori_loop(0, CH_PER_TILE, ebody, 0)
    pltpu.sync_copy(deg_v, out_hbm.at[w])


@functools.partial(
    pl.kernel,
    out_type=jax.ShapeDtypeStruct((NC, N_PAD, H), jnp.float32),
    mesh=_mesh,
    scratch_types=[
        pltpu.VMEM((CH_PER_TILE, CHUNK), jnp.int32),
        pltpu.VMEM((CH_PER_TILE, CHUNK), jnp.int32),
        [pltpu.VMEM((CHUNK, H), jnp.float32)] * KDEPTH,
        pltpu.VMEM_SHARED((N_PAD, H), jnp.float32),
        [pltpu.SemaphoreType.DMA] * KDEPTH,
    ],
)
def _sc_aggregate(u_hbm, src_hbm, dst_hbm, zero_hbm, out_hbm,
                  src_v, dst_v, bufs, acc, sems):
    c = lax.axis_index("c")
    s = lax.axis_index("s")
    w = c * NS + s
    pltpu.sync_copy(src_hbm.at[pl.ds(w * CH_PER_TILE, CH_PER_TILE)], src_v)
    pltpu.sync_copy(dst_hbm.at[pl.ds(w * CH_PER_TILE, CH_PER_TILE)], dst_v)
    pltpu.sync_copy(zero_hbm.at[pl.ds(s * ROWS_PER_TILE, ROWS_PER_TILE)],
                    acc.at[pl.ds(s * ROWS_PER_TILE, ROWS_PER_TILE)])
    plsc.subcore_barrier()

    def body(j, carry):
        base = j * KDEPTH
        cps = [
            pltpu.async_copy(u_hbm.at[src_v.at[base + i]], bufs[i], sems[i])
            for i in range(KDEPTH)
        ]
        for i in range(KDEPTH):
            cps[i].wait()
            pltpu.sync_copy(bufs[i], acc.at[dst_v.at[base + i]], add=True)
        return carry

    lax.fori_loop(0, CH_PER_TILE // KDEPTH, body, 0)
    plsc.subcore_barrier()
    pltpu.sync_copy(acc.at[pl.ds(s * ROWS_PER_TILE, ROWS_PER_TILE)],
                    out_hbm.at[c, pl.ds(s * ROWS_PER_TILE, ROWS_PER_TILE)])


def _tc_dinv(degp_ref, o_ref):
    deg = jnp.sum(degp_ref[...], axis=0, keepdims=True) + 1.0
    o_ref[...] = lax.rsqrt(deg)


def _tc_u1(x_ref, w_ref, dinv_ref, o_ref):
    o_ref[...] = jnp.dot(x_ref[...], w_ref[...],
                         preferred_element_type=jnp.float32) * dinv_ref[...]


def _tc_mid(a0_ref, a1_ref, u_ref, dinv_ref, b_ref, g_ref, be_ref, rm_ref,
            rv_ref, w2_ref, o_ref):
    dinv = dinv_ref[...]
    conv = dinv * (a0_ref[...] + a1_ref[...] + u_ref[...]) + b_ref[...]
    scale = g_ref[...] * lax.rsqrt(rv_ref[...] + EPS)
    h = jnp.maximum((conv - rm_ref[...]) * scale + be_ref[...], 0.0)
    o_ref[...] = jnp.dot(h, w2_ref[...],
                         preferred_element_type=jnp.float32) * dinv


def _tc_head(a0_ref, a1_ref, u_ref, dinv_ref, b_ref, g_ref, be_ref, rm_ref,
             rv_ref, batch_ref, l1w_ref, l1b_ref, l2w_ref, l2b_ref, o_ref):
    conv = dinv_ref[...] * (a0_ref[...] + a1_ref[...] + u_ref[...]) + b_ref[...]
    scale = g_ref[...] * lax.rsqrt(rv_ref[...] + EPS)
    h = jnp.maximum((conv - rm_ref[...]) * scale + be_ref[...], 0.0)
    seg = lax.broadcasted_iota(jnp.int32, (G, N_PAD), 0)
    ohT = (seg == batch_ref[...]).astype(jnp.float32)
    sums = jnp.dot(ohT, h, preferred_element_type=jnp.float32)
    cnt = jnp.dot(ohT, jnp.ones((N_PAD, 1), jnp.float32),
                  preferred_element_type=jnp.float32)
    pooled = sums / jnp.maximum(cnt, 1.0)
    z = jnp.maximum(jnp.dot(pooled, l1w_ref[...],
                            preferred_element_type=jnp.float32) + l1b_ref[...],
                    0.0)
    o_ref[...] = jnp.dot(z, l2w_ref[...],
                         preferred_element_type=jnp.float32) + l2b_ref[...]


def _call(body, out_shape, *args):
    return pl.pallas_call(body, out_shape=out_shape)(*args)


def kernel(x, edge_index, batch, W1, b1, W2, b2, g1, be1, rm1, rv1,
           g2, be2, rm2, rv2, l1w, l1b, l2w, l2b):
    f32 = jnp.float32
    src, dst = edge_index[0], edge_index[1]
    dummy = jnp.full((E_PAD - E,), N, jnp.int32)
    src_p = jnp.concatenate([src, dummy]).reshape(NW * CH_PER_TILE, CHUNK)
    dst_p = jnp.concatenate([dst, dummy]).reshape(NW * CH_PER_TILE, CHUNK)
    x_p = jnp.pad(x, ((0, N_PAD - N), (0, 0)))
    batch_row = jnp.pad(batch, (0, N_PAD - N), constant_values=G).reshape(1, N_PAD)
    zero2d = jnp.zeros((N_PAD, H), f32)

    w1t = W1.T
    w2t = W2.T
    l1wt = l1w.T
    l2wt = l2w.T
    row = lambda v: v.reshape(1, -1)

    deg_parts = _sc_degree(dst_p)
    dinv_row = _call(_tc_dinv,
                     jax.ShapeDtypeStruct((1, N_PAD), f32), deg_parts)
    dinv = dinv_row.reshape(N_PAD, 1)

    u1 = _call(_tc_u1, jax.ShapeDtypeStruct((N_PAD, H), f32),
               x_p, w1t, dinv)
    agg1 = _sc_aggregate(u1, src_p, dst_p, zero2d)
    u2 = _call(_tc_mid, jax.ShapeDtypeStruct((N_PAD, H), f32),
               agg1[0], agg1[1], u1, dinv, row(b1), row(g1), row(be1),
               row(rm1), row(rv1), w2t)
    agg2 = _sc_aggregate(u2, src_p, dst_p, zero2d)
    out = _call(_tc_head, jax.ShapeDtypeStruct((G, 1), f32),
                agg2[0], agg2[1], u2, dinv, row(b2), row(g2), row(be2),
                row(rm2), row(rv2), batch_row, l1wt, row(l1b), l2wt,
                l2b.reshape(1, 1))
    return out


# speedup vs baseline: 16.1808x; 16.1808x over previous
import functools

import jax
import jax.numpy as jnp
from jax import lax
from jax.experimental import pallas as pl
from jax.experimental.pallas import tpu as pltpu
from jax.experimental.pallas import tpu_sc as plsc

N = 10000
E = 320000
F_IN = 128
H = 64
G = 64
EPS = 1e-5

N_PAD = 10240
CHUNK = 128
NC, NS = 2, 16
NW = NC * NS
CH_PER_TILE = 80
E_PAD = NW * CH_PER_TILE * CHUNK
ROWS_PER_TILE = N_PAD // NS
KDEPTH = 4

_mesh = plsc.VectorSubcoreMesh(core_axis_name="c", subcore_axis_name="s")


DEGW = 16


@functools.partial(
    pl.kernel,
    out_type=jax.ShapeDtypeStruct((NC, N_PAD, DEGW), jnp.float32),
    mesh=_mesh,
    scratch_types=[
        pltpu.VMEM((CH_PER_TILE, CHUNK), jnp.int32),
        pltpu.VMEM((CHUNK, DEGW), jnp.float32),
        pltpu.VMEM_SHARED((N_PAD, DEGW), jnp.float32),
    ],
    compiler_params=pltpu.CompilerParams(use_tc_tiling_on_sc=False),
)
def _sc_degree(dst_hbm, ones_hbm, zero_hbm, out_hbm, idx_v, ones_v, acc):
    c = lax.axis_index("c")
    s = lax.axis_index("s")
    w = c * NS + s
    pltpu.sync_copy(dst_hbm.at[pl.ds(w * CH_PER_TILE, CH_PER_TILE)], idx_v)
    pltpu.sync_copy(ones_hbm, ones_v)
    pltpu.sync_copy(zero_hbm.at[pl.ds(s * ROWS_PER_TILE, ROWS_PER_TILE)],
                    acc.at[pl.ds(s * ROWS_PER_TILE, ROWS_PER_TILE)])
    plsc.subcore_barrier()

    def body(j, carry):
        pltpu.sync_copy(ones_v, acc.at[idx_v.at[j]], add=True)
        return carry

    lax.fori_loop(0, CH_PER_TILE, body, 0)
    plsc.subcore_barrier()
    pltpu.sync_copy(acc.at[pl.ds(s * ROWS_PER_TILE, ROWS_PER_TILE)],
                    out_hbm.at[c, pl.ds(s * ROWS_PER_TILE, ROWS_PER_TILE)])


@functools.partial(
    pl.kernel,
    out_type=jax.ShapeDtypeStruct((NC, N_PAD, H), jnp.float32),
    mesh=_mesh,
    scratch_types=[
        pltpu.VMEM((CH_PER_TILE, CHUNK), jnp.int32),
        pltpu.VMEM((CH_PER_TILE, CHUNK), jnp.int32),
        [pltpu.VMEM((CHUNK, H), jnp.float32)] * KDEPTH,
        pltpu.VMEM_SHARED((N_PAD, H), jnp.float32),
        [pltpu.SemaphoreType.DMA] * KDEPTH,
    ],
    compiler_params=pltpu.CompilerParams(use_tc_tiling_on_sc=False),
)
def _sc_aggregate(u_hbm, src_hbm, dst_hbm, zero_hbm, out_hbm,
                  src_v, dst_v, bufs, acc, sems):
    c = lax.axis_index("c")
    s = lax.axis_index("s")
    w = c * NS + s
    pltpu.sync_copy(src_hbm.at[pl.ds(w * CH_PER_TILE, CH_PER_TILE)], src_v)
    pltpu.sync_copy(dst_hbm.at[pl.ds(w * CH_PER_TILE, CH_PER_TILE)], dst_v)
    pltpu.sync_copy(zero_hbm.at[pl.ds(s * ROWS_PER_TILE, ROWS_PER_TILE)],
                    acc.at[pl.ds(s * ROWS_PER_TILE, ROWS_PER_TILE)])
    plsc.subcore_barrier()

    def body(j, carry):
        base = j * KDEPTH
        cps = [
            pltpu.async_copy(u_hbm.at[src_v.at[base + i]], bufs[i], sems[i])
            for i in range(KDEPTH)
        ]
        for i in range(KDEPTH):
            cps[i].wait()
            pltpu.sync_copy(bufs[i], acc.at[dst_v.at[base + i]], add=True)
        return carry

    lax.fori_loop(0, CH_PER_TILE // KDEPTH, body, 0)
    plsc.subcore_barrier()
    pltpu.sync_copy(acc.at[pl.ds(s * ROWS_PER_TILE, ROWS_PER_TILE)],
                    out_hbm.at[c, pl.ds(s * ROWS_PER_TILE, ROWS_PER_TILE)])


def _tc_dinv(degp_ref, o_ref):
    d = degp_ref[...]
    deg = d[0, :, 0:1] + d[1, :, 0:1] + 1.0
    o_ref[...] = lax.rsqrt(deg)


def _tc_u1(x_ref, w_ref, dinv_ref, o_ref):
    o_ref[...] = jnp.dot(x_ref[...], w_ref[...],
                         preferred_element_type=jnp.float32) * dinv_ref[...]


def _tc_mid(a0_ref, a1_ref, u_ref, dinv_ref, b_ref, g_ref, be_ref, rm_ref,
            rv_ref, w2_ref, o_ref):
    dinv = dinv_ref[...]
    conv = dinv * (a0_ref[...] + a1_ref[...] + u_ref[...]) + b_ref[...]
    scale = g_ref[...] * lax.rsqrt(rv_ref[...] + EPS)
    h = jnp.maximum((conv - rm_ref[...]) * scale + be_ref[...], 0.0)
    o_ref[...] = jnp.dot(h, w2_ref[...],
                         preferred_element_type=jnp.float32) * dinv


def _tc_head(a0_ref, a1_ref, u_ref, dinv_ref, b_ref, g_ref, be_ref, rm_ref,
             rv_ref, batch_ref, l1w_ref, l1b_ref, l2w_ref, l2b_ref, o_ref):
    conv = dinv_ref[...] * (a0_ref[...] + a1_ref[...] + u_ref[...]) + b_ref[...]
    scale = g_ref[...] * lax.rsqrt(rv_ref[...] + EPS)
    h = jnp.maximum((conv - rm_ref[...]) * scale + be_ref[...], 0.0)
    seg = lax.broadcasted_iota(jnp.int32, (G, N_PAD), 0)
    ohT = (seg == batch_ref[...]).astype(jnp.float32)
    sums = jnp.dot(ohT, h, preferred_element_type=jnp.float32)
    cnt = jnp.dot(ohT, jnp.ones((N_PAD, 1), jnp.float32),
                  preferred_element_type=jnp.float32)
    pooled = sums / jnp.maximum(cnt, 1.0)
    z = jnp.maximum(jnp.dot(pooled, l1w_ref[...],
                            preferred_element_type=jnp.float32) + l1b_ref[...],
                    0.0)
    o_ref[...] = jnp.dot(z, l2w_ref[...],
                         preferred_element_type=jnp.float32) + l2b_ref[...]


def _call(body, out_shape, *args):
    return pl.pallas_call(body, out_shape=out_shape)(*args)


def kernel(x, edge_index, batch, W1, b1, W2, b2, g1, be1, rm1, rv1,
           g2, be2, rm2, rv2, l1w, l1b, l2w, l2b):
    f32 = jnp.float32
    src, dst = edge_index[0], edge_index[1]
    dummy = jnp.full((E_PAD - E,), N, jnp.int32)
    src_p = jnp.concatenate([src, dummy]).reshape(NW * CH_PER_TILE, CHUNK)
    dst_p = jnp.concatenate([dst, dummy]).reshape(NW * CH_PER_TILE, CHUNK)
    x_p = jnp.pad(x, ((0, N_PAD - N), (0, 0)))
    batch_row = jnp.pad(batch, (0, N_PAD - N), constant_values=G).reshape(1, N_PAD)
    zero2d = jnp.zeros((N_PAD, H), f32)
    zero16 = jnp.zeros((N_PAD, DEGW), f32)
    ones16 = jnp.ones((CHUNK, DEGW), f32)

    w1t = W1.T
    w2t = W2.T
    l1wt = l1w.T
    l2wt = l2w.T
    row = lambda v: v.reshape(1, -1)

    deg_parts = _sc_degree(dst_p, ones16, zero16)
    dinv = _call(_tc_dinv,
                 jax.ShapeDtypeStruct((N_PAD, 1), f32), deg_parts)

    u1 = _call(_tc_u1, jax.ShapeDtypeStruct((N_PAD, H), f32),
               x_p, w1t, dinv)
    agg1 = _sc_aggregate(u1, src_p, dst_p, zero2d)
    u2 = _call(_tc_mid, jax.ShapeDtypeStruct((N_PAD, H), f32),
               agg1[0], agg1[1], u1, dinv, row(b1), row(g1), row(be1),
               row(rm1), row(rv1), w2t)
    agg2 = _sc_aggregate(u2, src_p, dst_p, zero2d)
    out = _call(_tc_head, jax.ShapeDtypeStruct((G, 1), f32),
                agg2[0], agg2[1], u2, dinv, row(b2), row(g2), row(be2),
                row(rm2), row(rv2), batch_row, l1wt, row(l1b), l2wt,
                l2b.reshape(1, 1))
    return out


# speedup vs baseline: 32.8997x; 2.0333x over previous
import functools

import jax
import jax.numpy as jnp
from jax import lax
from jax.experimental import pallas as pl
from jax.experimental.pallas import tpu as pltpu
from jax.experimental.pallas import tpu_sc as plsc

N = 10000
E = 320000
F_IN = 128
H = 64
G = 64
EPS = 1e-5

N_PAD = 10240
CHUNK = 128
NC, NS = 2, 16
NW = NC * NS
CH_PER_TILE = 80
E_PAD = NW * CH_PER_TILE * CHUNK
ROWS_PER_TILE = N_PAD // NS
KDEPTH = 4

_mesh = plsc.VectorSubcoreMesh(core_axis_name="c", subcore_axis_name="s")


DEGW = 16


@functools.partial(
    pl.kernel,
    out_type=jax.ShapeDtypeStruct((NC, N_PAD, DEGW), jnp.float32),
    mesh=_mesh,
    scratch_types=[
        pltpu.VMEM((CH_PER_TILE, CHUNK), jnp.int32),
        pltpu.VMEM((CHUNK, DEGW), jnp.float32),
        pltpu.VMEM_SHARED((N_PAD, DEGW), jnp.float32),
    ],
    compiler_params=pltpu.CompilerParams(use_tc_tiling_on_sc=False),
)
def _sc_degree(dst_hbm, ones_hbm, zero_hbm, out_hbm, idx_v, ones_v, acc):
    c = lax.axis_index("c")
    s = lax.axis_index("s")
    w = c * NS + s
    pltpu.sync_copy(dst_hbm.at[pl.ds(w * CH_PER_TILE, CH_PER_TILE)], idx_v)
    pltpu.sync_copy(ones_hbm, ones_v)
    pltpu.sync_copy(zero_hbm.at[pl.ds(s * ROWS_PER_TILE, ROWS_PER_TILE)],
                    acc.at[pl.ds(s * ROWS_PER_TILE, ROWS_PER_TILE)])
    plsc.subcore_barrier()

    def body(j, carry):
        pltpu.sync_copy(ones_v, acc.at[idx_v.at[j]], add=True)
        return carry

    lax.fori_loop(0, CH_PER_TILE, body, 0)
    plsc.subcore_barrier()
    pltpu.sync_copy(acc.at[pl.ds(s * ROWS_PER_TILE, ROWS_PER_TILE)],
                    out_hbm.at[c, pl.ds(s * ROWS_PER_TILE, ROWS_PER_TILE)])


@functools.partial(
    pl.kernel,
    out_type=jax.ShapeDtypeStruct((NC, N_PAD, H), jnp.float32),
    mesh=_mesh,
    scratch_types=[
        pltpu.VMEM((CH_PER_TILE, CHUNK), jnp.int32),
        pltpu.VMEM((CH_PER_TILE, CHUNK), jnp.int32),
        [pltpu.VMEM((CHUNK, H), jnp.float32)] * KDEPTH,
        pltpu.VMEM_SHARED((N_PAD, H), jnp.float32),
        [pltpu.SemaphoreType.DMA] * KDEPTH,
    ],
    compiler_params=pltpu.CompilerParams(use_tc_tiling_on_sc=False),
)
def _sc_aggregate(u_hbm, src_hbm, dst_hbm, zero_hbm, out_hbm,
                  src_v, dst_v, bufs, acc, sems):
    c = lax.axis_index("c")
    s = lax.axis_index("s")
    w = c * NS + s
    pltpu.sync_copy(src_hbm.at[pl.ds(w * CH_PER_TILE, CH_PER_TILE)], src_v)
    pltpu.sync_copy(dst_hbm.at[pl.ds(w * CH_PER_TILE, CH_PER_TILE)], dst_v)
    pltpu.sync_copy(zero_hbm.at[pl.ds(s * ROWS_PER_TILE, ROWS_PER_TILE)],
                    acc.at[pl.ds(s * ROWS_PER_TILE, ROWS_PER_TILE)])
    plsc.subcore_barrier()

    def body(j, carry):
        base = j * KDEPTH
        cps = [
            pltpu.async_copy(u_hbm.at[src_v.at[base + i]], bufs[i], sems[i])
            for i in range(KDEPTH)
        ]
        for i in range(KDEPTH):
            cps[i].wait()
            pltpu.sync_copy(bufs[i], acc.at[dst_v.at[base + i]], add=True)
        return carry

    lax.fori_loop(0, CH_PER_TILE // KDEPTH, body, 0)
    plsc.subcore_barrier()
    pltpu.sync_copy(acc.at[pl.ds(s * ROWS_PER_TILE, ROWS_PER_TILE)],
                    out_hbm.at[c, pl.ds(s * ROWS_PER_TILE, ROWS_PER_TILE)])


def _tc_dinv(degp_ref, o_ref):
    d = degp_ref[...]
    deg = d[0, :, 0:1] + d[1, :, 0:1] + 1.0
    o_ref[...] = lax.rsqrt(deg)


def _tc_u1(x_ref, w_ref, dinv_ref, o_ref):
    o_ref[...] = jnp.dot(x_ref[...], w_ref[...],
                         preferred_element_type=jnp.float32) * dinv_ref[...]


def _tc_mid(a0_ref, a1_ref, u_ref, dinv_ref, b_ref, g_ref, be_ref, rm_ref,
            rv_ref, w2_ref, o_ref):
    dinv = dinv_ref[...]
    conv = dinv * (a0_ref[...] + a1_ref[...] + u_ref[...]) + b_ref[...]
    scale = g_ref[...] * lax.rsqrt(rv_ref[...] + EPS)
    h = jnp.maximum((conv - rm_ref[...]) * scale + be_ref[...], 0.0)
    o_ref[...] = jnp.dot(h, w2_ref[...],
                         preferred_element_type=jnp.float32) * dinv


def _tc_head(a0_ref, a1_ref, u_ref, dinv_ref, b_ref, g_ref, be_ref, rm_ref,
             rv_ref, batch_ref, l1w_ref, l1b_ref, l2w_ref, l2b_ref, o_ref):
    conv = dinv_ref[...] * (a0_ref[...] + a1_ref[...] + u_ref[...]) + b_ref[...]
    scale = g_ref[...] * lax.rsqrt(rv_ref[...] + EPS)
    h = jnp.maximum((conv - rm_ref[...]) * scale + be_ref[...], 0.0)
    seg = lax.broadcasted_iota(jnp.int32, (G, N_PAD), 0)
    ohT = (seg == batch_ref[...]).astype(jnp.float32)
    sums = jnp.dot(ohT, h, preferred_element_type=jnp.float32)
    cnt = jnp.dot(ohT, jnp.ones((N_PAD, 1), jnp.float32),
                  preferred_element_type=jnp.float32)
    pooled = sums / jnp.maximum(cnt, 1.0)
    z = jnp.maximum(jnp.dot(pooled, l1w_ref[...],
                            preferred_element_type=jnp.float32) + l1b_ref[...],
                    0.0)
    o_ref[...] = jnp.dot(z, l2w_ref[...],
                         preferred_element_type=jnp.float32) + l2b_ref[...]


def _call(body, out_shape, *args):
    return pl.pallas_call(body, out_shape=out_shape)(*args)


def kernel(x, edge_index, batch, W1, b1, W2, b2, g1, be1, rm1, rv1,
           g2, be2, rm2, rv2, l1w, l1b, l2w, l2b):
    f32 = jnp.float32
    src, dst = edge_index[0], edge_index[1]
    dummy = N + jnp.arange(E_PAD - E, dtype=jnp.int32) % (N_PAD - N)
    src_p = jnp.concatenate([src, dummy]).reshape(NW * CH_PER_TILE, CHUNK)
    dst_p = jnp.concatenate([dst, dummy]).reshape(NW * CH_PER_TILE, CHUNK)
    x_p = jnp.pad(x, ((0, N_PAD - N), (0, 0)))
    batch_row = jnp.pad(batch, (0, N_PAD - N), constant_values=G).reshape(1, N_PAD)
    zero2d = jnp.zeros((N_PAD, H), f32)
    zero16 = jnp.zeros((N_PAD, DEGW), f32)
    ones16 = jnp.ones((CHUNK, DEGW), f32)

    w1t = W1.T
    w2t = W2.T
    l1wt = l1w.T
    l2wt = l2w.T
    row = lambda v: v.reshape(1, -1)

    deg_parts = _sc_degree(dst_p, ones16, zero16)
    dinv = _call(_tc_dinv,
                 jax.ShapeDtypeStruct((N_PAD, 1), f32), deg_parts)

    u1 = _call(_tc_u1, jax.ShapeDtypeStruct((N_PAD, H), f32),
               x_p, w1t, dinv)
    agg1 = _sc_aggregate(u1, src_p, dst_p, zero2d)
    u2 = _call(_tc_mid, jax.ShapeDtypeStruct((N_PAD, H), f32),
               agg1[0], agg1[1], u1, dinv, row(b1), row(g1), row(be1),
               row(rm1), row(rv1), w2t)
    agg2 = _sc_aggregate(u2, src_p, dst_p, zero2d)
    out = _call(_tc_head, jax.ShapeDtypeStruct((G, 1), f32),
                agg2[0], agg2[1], u2, dinv, row(b2), row(g2), row(be2),
                row(rm2), row(rv2), batch_row, l1wt, row(l1b), l2wt,
                l2b.reshape(1, 1))
    return out


# speedup vs baseline: 37.8859x; 1.1516x over previous
import functools

import jax
import jax.numpy as jnp
from jax import lax
from jax.experimental import pallas as pl
from jax.experimental.pallas import tpu as pltpu
from jax.experimental.pallas import tpu_sc as plsc

N = 10000
E = 320000
F_IN = 128
H = 64
G = 64
EPS = 1e-5

N_PAD = 10240
CHUNK = 128
NC, NS = 2, 16
NW = NC * NS
CH_PER_TILE = 80
E_PAD = NW * CH_PER_TILE * CHUNK
ROWS_PER_TILE = N_PAD // NS
KDEPTH = 8

_mesh = plsc.VectorSubcoreMesh(core_axis_name="c", subcore_axis_name="s")


DEGW = 16


@functools.partial(
    pl.kernel,
    out_type=jax.ShapeDtypeStruct((NC, N_PAD, DEGW), jnp.float32),
    mesh=_mesh,
    scratch_types=[
        pltpu.VMEM((CH_PER_TILE, CHUNK), jnp.int32),
        pltpu.VMEM((CHUNK, DEGW), jnp.float32),
        pltpu.VMEM_SHARED((N_PAD, DEGW), jnp.float32),
    ],
    compiler_params=pltpu.CompilerParams(use_tc_tiling_on_sc=False),
)
def _sc_degree(dst_hbm, ones_hbm, zero_hbm, out_hbm, idx_v, ones_v, acc):
    c = lax.axis_index("c")
    s = lax.axis_index("s")
    w = c * NS + s
    pltpu.sync_copy(dst_hbm.at[pl.ds(w * CH_PER_TILE, CH_PER_TILE)], idx_v)
    pltpu.sync_copy(ones_hbm, ones_v)
    pltpu.sync_copy(zero_hbm.at[pl.ds(s * ROWS_PER_TILE, ROWS_PER_TILE)],
                    acc.at[pl.ds(s * ROWS_PER_TILE, ROWS_PER_TILE)])
    plsc.subcore_barrier()

    def body(j, carry):
        pltpu.sync_copy(ones_v, acc.at[idx_v.at[j]], add=True)
        return carry

    lax.fori_loop(0, CH_PER_TILE, body, 0)
    plsc.subcore_barrier()
    pltpu.sync_copy(acc.at[pl.ds(s * ROWS_PER_TILE, ROWS_PER_TILE)],
                    out_hbm.at[c, pl.ds(s * ROWS_PER_TILE, ROWS_PER_TILE)])


@functools.partial(
    pl.kernel,
    out_type=jax.ShapeDtypeStruct((NC, N_PAD, H), jnp.float32),
    mesh=_mesh,
    scratch_types=[
        pltpu.VMEM((CH_PER_TILE + KDEPTH, CHUNK), jnp.int32),
        pltpu.VMEM((CH_PER_TILE, CHUNK), jnp.int32),
        [pltpu.VMEM((CHUNK, H), jnp.float32)] * KDEPTH,
        pltpu.VMEM_SHARED((N_PAD, H), jnp.float32),
        [pltpu.SemaphoreType.DMA] * KDEPTH,
        [pltpu.SemaphoreType.DMA] * KDEPTH,
    ],
    compiler_params=pltpu.CompilerParams(use_tc_tiling_on_sc=False),
)
def _sc_aggregate(u_hbm, src_hbm, dst_hbm, zero_hbm, out_hbm,
                  src_v, dst_v, bufs, acc, gsems, ssems):
    c = lax.axis_index("c")
    s = lax.axis_index("s")
    w = c * NS + s
    pltpu.sync_copy(src_hbm.at[pl.ds(w * CH_PER_TILE, CH_PER_TILE)],
                    src_v.at[pl.ds(0, CH_PER_TILE)])
    pltpu.sync_copy(src_hbm.at[pl.ds(w * CH_PER_TILE, KDEPTH)],
                    src_v.at[pl.ds(CH_PER_TILE, KDEPTH)])
    pltpu.sync_copy(dst_hbm.at[pl.ds(w * CH_PER_TILE, CH_PER_TILE)], dst_v)
    pltpu.sync_copy(zero_hbm.at[pl.ds(s * ROWS_PER_TILE, ROWS_PER_TILE)],
                    acc.at[pl.ds(s * ROWS_PER_TILE, ROWS_PER_TILE)])
    gprime = [
        pltpu.async_copy(u_hbm.at[src_v.at[i]], bufs[i], gsems[i])
        for i in range(KDEPTH)
    ]
    plsc.subcore_barrier()

    def body(j, carry):
        base = j * KDEPTH
        scats = []
        for i in range(KDEPTH):
            pltpu.make_async_copy(u_hbm.at[src_v.at[base + i]], bufs[i],
                                  gsems[i]).wait()
            scats.append(pltpu.async_copy(
                bufs[i], acc.at[dst_v.at[base + i]], ssems[i], add=True))
        for i in range(KDEPTH):
            scats[i].wait()
            pltpu.async_copy(u_hbm.at[src_v.at[base + KDEPTH + i]], bufs[i],
                             gsems[i])
        return carry

    lax.fori_loop(0, CH_PER_TILE // KDEPTH, body, 0)
    for i in range(KDEPTH):
        pltpu.make_async_copy(u_hbm.at[src_v.at[CH_PER_TILE + i]], bufs[i],
                              gsems[i]).wait()
    plsc.subcore_barrier()
    pltpu.sync_copy(acc.at[pl.ds(s * ROWS_PER_TILE, ROWS_PER_TILE)],
                    out_hbm.at[c, pl.ds(s * ROWS_PER_TILE, ROWS_PER_TILE)])


def _tc_dinv(degp_ref, o_ref):
    d = degp_ref[...]
    deg = d[0, :, 0:1] + d[1, :, 0:1] + 1.0
    o_ref[...] = lax.rsqrt(deg)


def _tc_u1(x_ref, w_ref, dinv_ref, o_ref):
    o_ref[...] = jnp.dot(x_ref[...], w_ref[...],
                         preferred_element_type=jnp.float32) * dinv_ref[...]


def _tc_mid(a0_ref, a1_ref, u_ref, dinv_ref, b_ref, g_ref, be_ref, rm_ref,
            rv_ref, w2_ref, o_ref):
    dinv = dinv_ref[...]
    conv = dinv * (a0_ref[...] + a1_ref[...] + u_ref[...]) + b_ref[...]
    scale = g_ref[...] * lax.rsqrt(rv_ref[...] + EPS)
    h = jnp.maximum((conv - rm_ref[...]) * scale + be_ref[...], 0.0)
    o_ref[...] = jnp.dot(h, w2_ref[...],
                         preferred_element_type=jnp.float32) * dinv


def _tc_head(a0_ref, a1_ref, u_ref, dinv_ref, b_ref, g_ref, be_ref, rm_ref,
             rv_ref, batch_ref, l1w_ref, l1b_ref, l2w_ref, l2b_ref, o_ref):
    conv = dinv_ref[...] * (a0_ref[...] + a1_ref[...] + u_ref[...]) + b_ref[...]
    scale = g_ref[...] * lax.rsqrt(rv_ref[...] + EPS)
    h = jnp.maximum((conv - rm_ref[...]) * scale + be_ref[...], 0.0)
    seg = lax.broadcasted_iota(jnp.int32, (G, N_PAD), 0)
    ohT = (seg == batch_ref[...]).astype(jnp.float32)
    sums = jnp.dot(ohT, h, preferred_element_type=jnp.float32)
    cnt = jnp.dot(ohT, jnp.ones((N_PAD, 1), jnp.float32),
                  preferred_element_type=jnp.float32)
    pooled = sums / jnp.maximum(cnt, 1.0)
    z = jnp.maximum(jnp.dot(pooled, l1w_ref[...],
                            preferred_element_type=jnp.float32) + l1b_ref[...],
                    0.0)
    o_ref[...] = jnp.dot(z, l2w_ref[...],
                         preferred_element_type=jnp.float32) + l2b_ref[...]


def _call(body, out_shape, *args):
    return pl.pallas_call(body, out_shape=out_shape)(*args)


def kernel(x, edge_index, batch, W1, b1, W2, b2, g1, be1, rm1, rv1,
           g2, be2, rm2, rv2, l1w, l1b, l2w, l2b):
    f32 = jnp.float32
    src, dst = edge_index[0], edge_index[1]
    dummy = N + jnp.arange(E_PAD - E, dtype=jnp.int32) % (N_PAD - N)
    src_p = jnp.concatenate([src, dummy]).reshape(NW * CH_PER_TILE, CHUNK)
    dst_p = jnp.concatenate([dst, dummy]).reshape(NW * CH_PER_TILE, CHUNK)
    x_p = jnp.pad(x, ((0, N_PAD - N), (0, 0)))
    batch_row = jnp.pad(batch, (0, N_PAD - N), constant_values=G).reshape(1, N_PAD)
    zero2d = jnp.zeros((N_PAD, H), f32)
    zero16 = jnp.zeros((N_PAD, DEGW), f32)
    ones16 = jnp.ones((CHUNK, DEGW), f32)

    w1t = W1.T
    w2t = W2.T
    l1wt = l1w.T
    l2wt = l2w.T
    row = lambda v: v.reshape(1, -1)

    deg_parts = _sc_degree(dst_p, ones16, zero16)
    dinv = _call(_tc_dinv,
                 jax.ShapeDtypeStruct((N_PAD, 1), f32), deg_parts)

    u1 = _call(_tc_u1, jax.ShapeDtypeStruct((N_PAD, H), f32),
               x_p, w1t, dinv)
    agg1 = _sc_aggregate(u1, src_p, dst_p, zero2d)
    u2 = _call(_tc_mid, jax.ShapeDtypeStruct((N_PAD, H), f32),
               agg1[0], agg1[1], u1, dinv, row(b1), row(g1), row(be1),
               row(rm1), row(rv1), w2t)
    agg2 = _sc_aggregate(u2, src_p, dst_p, zero2d)
    out = _call(_tc_head, jax.ShapeDtypeStruct((G, 1), f32),
                agg2[0], agg2[1], u2, dinv, row(b2), row(g2), row(be2),
                row(rm2), row(rv2), batch_row, l1wt, row(l1b), l2wt,
                l2b.reshape(1, 1))
    return out


# speedup vs baseline: 40.1648x; 1.0602x over previous
import functools

import jax
import jax.numpy as jnp
from jax import lax
from jax.experimental import pallas as pl
from jax.experimental.pallas import tpu as pltpu
from jax.experimental.pallas import tpu_sc as plsc

N = 10000
E = 320000
F_IN = 128
H = 64
G = 64
EPS = 1e-5

N_PAD = 10240
CHUNK = 128
NC, NS = 2, 16
NW = NC * NS
CH_PER_TILE = 80
E_PAD = NW * CH_PER_TILE * CHUNK
ROWS_PER_TILE = N_PAD // NS
KDEPTH = 8

_mesh = plsc.VectorSubcoreMesh(core_axis_name="c", subcore_axis_name="s")


DEGW = 16


@functools.partial(
    pl.kernel,
    out_type=jax.ShapeDtypeStruct((NC, N_PAD, DEGW), jnp.float32),
    mesh=_mesh,
    scratch_types=[
        pltpu.VMEM((CH_PER_TILE, CHUNK), jnp.int32),
        pltpu.VMEM((CHUNK, DEGW), jnp.float32),
        pltpu.VMEM((CHUNK, DEGW), jnp.float32),
        pltpu.VMEM_SHARED((N_PAD, DEGW), jnp.float32),
    ],
    compiler_params=pltpu.CompilerParams(use_tc_tiling_on_sc=False),
)
def _sc_degree(dst_hbm, out_hbm, idx_v, ones_v, zero_v, acc):
    c = lax.axis_index("c")
    s = lax.axis_index("s")
    w = c * NS + s
    pltpu.sync_copy(dst_hbm.at[pl.ds(w * CH_PER_TILE, CH_PER_TILE)], idx_v)

    def fill(i, carry):
        ones_v[i, :] = jnp.full((DEGW,), 1.0, jnp.float32)
        zero_v[i, :] = jnp.zeros((DEGW,), jnp.float32)
        return carry

    lax.fori_loop(0, CHUNK, fill, 0)
    for k in range(ROWS_PER_TILE // CHUNK):
        pltpu.sync_copy(zero_v,
                        acc.at[pl.ds(s * ROWS_PER_TILE + k * CHUNK, CHUNK)])
    plsc.subcore_barrier()

    def body(j, carry):
        pltpu.sync_copy(ones_v, acc.at[idx_v.at[j]], add=True)
        return carry

    lax.fori_loop(0, CH_PER_TILE, body, 0)
    plsc.subcore_barrier()
    pltpu.sync_copy(acc.at[pl.ds(s * ROWS_PER_TILE, ROWS_PER_TILE)],
                    out_hbm.at[c, pl.ds(s * ROWS_PER_TILE, ROWS_PER_TILE)])


@functools.partial(
    pl.kernel,
    out_type=jax.ShapeDtypeStruct((NC, N_PAD, H), jnp.float32),
    mesh=_mesh,
    scratch_types=[
        pltpu.VMEM((CH_PER_TILE + KDEPTH, CHUNK), jnp.int32),
        pltpu.VMEM((CH_PER_TILE, CHUNK), jnp.int32),
        [pltpu.VMEM((CHUNK, H), jnp.float32)] * KDEPTH,
        pltpu.VMEM_SHARED((N_PAD, H), jnp.float32),
        [pltpu.SemaphoreType.DMA] * KDEPTH,
        [pltpu.SemaphoreType.DMA] * KDEPTH,
    ],
    compiler_params=pltpu.CompilerParams(use_tc_tiling_on_sc=False),
)
def _sc_aggregate(u_hbm, src_hbm, dst_hbm, zero_hbm, out_hbm,
                  src_v, dst_v, bufs, acc, gsems, ssems):
    c = lax.axis_index("c")
    s = lax.axis_index("s")
    w = c * NS + s
    pltpu.sync_copy(src_hbm.at[pl.ds(w * CH_PER_TILE, CH_PER_TILE)],
                    src_v.at[pl.ds(0, CH_PER_TILE)])
    pltpu.sync_copy(src_hbm.at[pl.ds(w * CH_PER_TILE, KDEPTH)],
                    src_v.at[pl.ds(CH_PER_TILE, KDEPTH)])
    pltpu.sync_copy(dst_hbm.at[pl.ds(w * CH_PER_TILE, CH_PER_TILE)], dst_v)
    pltpu.sync_copy(zero_hbm.at[pl.ds(s * ROWS_PER_TILE, ROWS_PER_TILE)],
                    acc.at[pl.ds(s * ROWS_PER_TILE, ROWS_PER_TILE)])
    gprime = [
        pltpu.async_copy(u_hbm.at[src_v.at[i]], bufs[i], gsems[i])
        for i in range(KDEPTH)
    ]
    plsc.subcore_barrier()

    def body(j, carry):
        base = j * KDEPTH
        scats = []
        for i in range(KDEPTH):
            pltpu.make_async_copy(u_hbm.at[src_v.at[base + i]], bufs[i],
                                  gsems[i]).wait()
            scats.append(pltpu.async_copy(
                bufs[i], acc.at[dst_v.at[base + i]], ssems[i], add=True))
        for i in range(KDEPTH):
            scats[i].wait()
            pltpu.async_copy(u_hbm.at[src_v.at[base + KDEPTH + i]], bufs[i],
                             gsems[i])
        return carry

    lax.fori_loop(0, CH_PER_TILE // KDEPTH, body, 0)
    for i in range(KDEPTH):
        pltpu.make_async_copy(u_hbm.at[src_v.at[CH_PER_TILE + i]], bufs[i],
                              gsems[i]).wait()
    plsc.subcore_barrier()
    pltpu.sync_copy(acc.at[pl.ds(s * ROWS_PER_TILE, ROWS_PER_TILE)],
                    out_hbm.at[c, pl.ds(s * ROWS_PER_TILE, ROWS_PER_TILE)])


def _tc_lin1(x_ref, w_ref, o_ref):
    o_ref[...] = jnp.dot(x_ref[...], w_ref[...],
                         preferred_element_type=jnp.float32)


def _tc_scale(degp_ref, lin_ref, u_ref, dinv_ref):
    d = degp_ref[...]
    deg = d[0, :, 0:1] + d[1, :, 0:1] + 1.0
    dinv = lax.rsqrt(deg)
    dinv_ref[...] = dinv
    u_ref[...] = lin_ref[...] * dinv


def _tc_mid(a0_ref, a1_ref, u_ref, dinv_ref, b_ref, g_ref, be_ref, rm_ref,
            rv_ref, w2_ref, o_ref):
    dinv = dinv_ref[...]
    conv = dinv * (a0_ref[...] + a1_ref[...] + u_ref[...]) + b_ref[...]
    scale = g_ref[...] * lax.rsqrt(rv_ref[...] + EPS)
    h = jnp.maximum((conv - rm_ref[...]) * scale + be_ref[...], 0.0)
    o_ref[...] = jnp.dot(h, w2_ref[...],
                         preferred_element_type=jnp.float32) * dinv


def _tc_head(a0_ref, a1_ref, u_ref, dinv_ref, b_ref, g_ref, be_ref, rm_ref,
             rv_ref, batch_ref, l1w_ref, l1b_ref, l2w_ref, l2b_ref, o_ref):
    conv = dinv_ref[...] * (a0_ref[...] + a1_ref[...] + u_ref[...]) + b_ref[...]
    scale = g_ref[...] * lax.rsqrt(rv_ref[...] + EPS)
    h = jnp.maximum((conv - rm_ref[...]) * scale + be_ref[...], 0.0)
    seg = lax.broadcasted_iota(jnp.int32, (G, N_PAD), 0)
    ohT = (seg == batch_ref[...]).astype(jnp.float32)
    sums = jnp.dot(ohT, h, preferred_element_type=jnp.float32)
    cnt = jnp.dot(ohT, jnp.ones((N_PAD, 1), jnp.float32),
                  preferred_element_type=jnp.float32)
    pooled = sums / jnp.maximum(cnt, 1.0)
    z = jnp.maximum(jnp.dot(pooled, l1w_ref[...],
                            preferred_element_type=jnp.float32) + l1b_ref[...],
                    0.0)
    o_ref[...] = jnp.dot(z, l2w_ref[...],
                         preferred_element_type=jnp.float32) + l2b_ref[...]


def _call(body, out_shape, *args):
    return pl.pallas_call(body, out_shape=out_shape)(*args)


def kernel(x, edge_index, batch, W1, b1, W2, b2, g1, be1, rm1, rv1,
           g2, be2, rm2, rv2, l1w, l1b, l2w, l2b):
    f32 = jnp.float32
    src, dst = edge_index[0], edge_index[1]
    dummy = N + jnp.arange(E_PAD - E, dtype=jnp.int32) % (N_PAD - N)
    ei_p = jnp.concatenate([edge_index, jnp.tile(dummy, (2, 1))], axis=1)
    src_p = ei_p[0].reshape(NW * CH_PER_TILE, CHUNK)
    dst_p = ei_p[1].reshape(NW * CH_PER_TILE, CHUNK)
    x_p = jnp.pad(x, ((0, N_PAD - N), (0, 0)))
    zero2d = jnp.zeros((N_PAD, H), f32)
    batch_row = jnp.pad(batch, (0, N_PAD - N), constant_values=G).reshape(1, N_PAD)

    w1t = W1.T
    w2t = W2.T
    l1wt = l1w.T
    l2wt = l2w.T
    row = lambda v: v.reshape(1, -1)

    deg_parts = _sc_degree(dst_p)
    lin1 = _call(_tc_lin1, jax.ShapeDtypeStruct((N_PAD, H), f32),
                 x_p, w1t)
    u1, dinv = _call(_tc_scale,
                     (jax.ShapeDtypeStruct((N_PAD, H), f32),
                      jax.ShapeDtypeStruct((N_PAD, 1), f32)),
                     deg_parts, lin1)
    agg1 = _sc_aggregate(u1, src_p, dst_p, zero2d)
    u2 = _call(_tc_mid, jax.ShapeDtypeStruct((N_PAD, H), f32),
               agg1[0], agg1[1], u1, dinv, row(b1), row(g1), row(be1),
               row(rm1), row(rv1), w2t)
    agg2 = _sc_aggregate(u2, src_p, dst_p, zero2d)
    out = _call(_tc_head, jax.ShapeDtypeStruct((G, 1), f32),
                agg2[0], agg2[1], u2, dinv, row(b2), row(g2), row(be2),
                row(rm2), row(rv2), batch_row, l1wt, row(l1b), l2wt,
                l2b.reshape(1, 1))
    return out


# speedup vs baseline: 41.0405x; 1.0218x over previous
import functools

import jax
import jax.numpy as jnp
from jax import lax
from jax.experimental import pallas as pl
from jax.experimental.pallas import tpu as pltpu
from jax.experimental.pallas import tpu_sc as plsc

N = 10000
E = 320000
F_IN = 128
H = 64
G = 64
EPS = 1e-5

N_PAD = 10240
CHUNK = 128
NC, NS = 2, 16
NW = NC * NS
CH_PER_TILE = 80
E_PAD = NW * CH_PER_TILE * CHUNK
ROWS_PER_TILE = N_PAD // NS
KDEPTH = 8

_mesh = plsc.VectorSubcoreMesh(core_axis_name="c", subcore_axis_name="s")


DEGW = 16


@functools.partial(
    pl.kernel,
    out_type=jax.ShapeDtypeStruct((NC, N_PAD, DEGW), jnp.float32),
    mesh=_mesh,
    scratch_types=[
        pltpu.VMEM((CH_PER_TILE, CHUNK), jnp.int32),
        pltpu.VMEM((CHUNK, DEGW), jnp.float32),
        pltpu.VMEM((CHUNK, DEGW), jnp.float32),
        pltpu.VMEM_SHARED((N_PAD, DEGW), jnp.float32),
    ],
    compiler_params=pltpu.CompilerParams(use_tc_tiling_on_sc=False),
)
def _sc_degree(dst_hbm, out_hbm, idx_v, ones_v, zero_v, acc):
    c = lax.axis_index("c")
    s = lax.axis_index("s")
    w = c * NS + s
    pltpu.sync_copy(dst_hbm.at[pl.ds(w * CH_PER_TILE, CH_PER_TILE)], idx_v)

    def fill(i, carry):
        ones_v[i, :] = jnp.full((DEGW,), 1.0, jnp.float32)
        zero_v[i, :] = jnp.zeros((DEGW,), jnp.float32)
        return carry

    lax.fori_loop(0, CHUNK, fill, 0)
    for k in range(ROWS_PER_TILE // CHUNK):
        pltpu.sync_copy(zero_v,
                        acc.at[pl.ds(s * ROWS_PER_TILE + k * CHUNK, CHUNK)])
    plsc.subcore_barrier()

    def body(j, carry):
        pltpu.sync_copy(ones_v, acc.at[idx_v.at[j]], add=True)
        return carry

    lax.fori_loop(0, CH_PER_TILE, body, 0)
    plsc.subcore_barrier()
    pltpu.sync_copy(acc.at[pl.ds(s * ROWS_PER_TILE, ROWS_PER_TILE)],
                    out_hbm.at[c, pl.ds(s * ROWS_PER_TILE, ROWS_PER_TILE)])


@functools.partial(
    pl.kernel,
    out_type=jax.ShapeDtypeStruct((NC, N_PAD, H), jnp.float32),
    mesh=_mesh,
    scratch_types=[
        pltpu.VMEM((CH_PER_TILE + KDEPTH, CHUNK), jnp.int32),
        pltpu.VMEM((CH_PER_TILE, CHUNK), jnp.int32),
        [pltpu.VMEM((CHUNK, H), jnp.float32)] * KDEPTH,
        pltpu.VMEM_SHARED((N_PAD, H), jnp.float32),
        [pltpu.SemaphoreType.DMA] * KDEPTH,
        [pltpu.SemaphoreType.DMA] * KDEPTH,
    ],
    compiler_params=pltpu.CompilerParams(use_tc_tiling_on_sc=False),
)
def _sc_aggregate(u_hbm, src_hbm, dst_hbm, out_hbm,
                  src_v, dst_v, bufs, acc, gsems, ssems):
    c = lax.axis_index("c")
    s = lax.axis_index("s")
    w = c * NS + s
    pltpu.sync_copy(src_hbm.at[pl.ds(w * CH_PER_TILE, CH_PER_TILE)],
                    src_v.at[pl.ds(0, CH_PER_TILE)])
    pltpu.sync_copy(src_hbm.at[pl.ds(w * CH_PER_TILE, KDEPTH)],
                    src_v.at[pl.ds(CH_PER_TILE, KDEPTH)])
    pltpu.sync_copy(dst_hbm.at[pl.ds(w * CH_PER_TILE, CH_PER_TILE)], dst_v)

    def zfill(i, carry):
        for q in range(H // 16):
            bufs[0][i, pl.ds(q * 16, 16)] = jnp.zeros((16,), jnp.float32)
        return carry

    lax.fori_loop(0, CHUNK, zfill, 0)
    for k in range(ROWS_PER_TILE // CHUNK):
        pltpu.sync_copy(bufs[0],
                        acc.at[pl.ds(s * ROWS_PER_TILE + k * CHUNK, CHUNK)])
    gprime = [
        pltpu.async_copy(u_hbm.at[src_v.at[i]], bufs[i], gsems[i])
        for i in range(KDEPTH)
    ]
    plsc.subcore_barrier()

    def body(j, carry):
        base = j * KDEPTH
        scats = []
        for i in range(KDEPTH):
            pltpu.make_async_copy(u_hbm.at[src_v.at[base + i]], bufs[i],
                                  gsems[i]).wait()
            scats.append(pltpu.async_copy(
                bufs[i], acc.at[dst_v.at[base + i]], ssems[i], add=True))
        for i in range(KDEPTH):
            scats[i].wait()
            pltpu.async_copy(u_hbm.at[src_v.at[base + KDEPTH + i]], bufs[i],
                             gsems[i])
        return carry

    lax.fori_loop(0, CH_PER_TILE // KDEPTH, body, 0)
    for i in range(KDEPTH):
        pltpu.make_async_copy(u_hbm.at[src_v.at[CH_PER_TILE + i]], bufs[i],
                              gsems[i]).wait()
    plsc.subcore_barrier()
    pltpu.sync_copy(acc.at[pl.ds(s * ROWS_PER_TILE, ROWS_PER_TILE)],
                    out_hbm.at[c, pl.ds(s * ROWS_PER_TILE, ROWS_PER_TILE)])


def _tc_lin1(x_ref, w_ref, o_ref):
    o_ref[...] = jnp.dot(x_ref[...], w_ref[...],
                         preferred_element_type=jnp.float32)


def _tc_scale(degp_ref, lin_ref, u_ref, dinv_ref):
    d = degp_ref[...]
    deg = d[0, :, 0:1] + d[1, :, 0:1] + 1.0
    dinv = lax.rsqrt(deg)
    dinv_ref[...] = dinv
    u_ref[...] = lin_ref[...] * dinv


def _tc_mid(a0_ref, a1_ref, u_ref, dinv_ref, b_ref, g_ref, be_ref, rm_ref,
            rv_ref, w2_ref, o_ref):
    dinv = dinv_ref[...]
    conv = dinv * (a0_ref[...] + a1_ref[...] + u_ref[...]) + b_ref[...]
    scale = g_ref[...] * lax.rsqrt(rv_ref[...] + EPS)
    h = jnp.maximum((conv - rm_ref[...]) * scale + be_ref[...], 0.0)
    o_ref[...] = jnp.dot(h, w2_ref[...],
                         preferred_element_type=jnp.float32) * dinv


def _tc_head(a0_ref, a1_ref, u_ref, dinv_ref, b_ref, g_ref, be_ref, rm_ref,
             rv_ref, batch_ref, l1w_ref, l1b_ref, l2w_ref, l2b_ref, o_ref):
    conv = dinv_ref[...] * (a0_ref[...] + a1_ref[...] + u_ref[...]) + b_ref[...]
    scale = g_ref[...] * lax.rsqrt(rv_ref[...] + EPS)
    h = jnp.maximum((conv - rm_ref[...]) * scale + be_ref[...], 0.0)
    seg = lax.broadcasted_iota(jnp.int32, (G, N_PAD), 0)
    ohT = (seg == batch_ref[...]).astype(jnp.float32)
    sums = jnp.dot(ohT, h, preferred_element_type=jnp.float32)
    cnt = jnp.dot(ohT, jnp.ones((N_PAD, 1), jnp.float32),
                  preferred_element_type=jnp.float32)
    pooled = sums / jnp.maximum(cnt, 1.0)
    z = jnp.maximum(jnp.dot(pooled, l1w_ref[...],
                            preferred_element_type=jnp.float32) + l1b_ref[...],
                    0.0)
    o_ref[...] = jnp.dot(z, l2w_ref[...],
                         preferred_element_type=jnp.float32) + l2b_ref[...]


def _call(body, out_shape, *args):
    return pl.pallas_call(body, out_shape=out_shape)(*args)


def kernel(x, edge_index, batch, W1, b1, W2, b2, g1, be1, rm1, rv1,
           g2, be2, rm2, rv2, l1w, l1b, l2w, l2b):
    f32 = jnp.float32
    src, dst = edge_index[0], edge_index[1]
    dummy = N + jnp.arange(E_PAD - E, dtype=jnp.int32) % (N_PAD - N)
    ei_p = jnp.concatenate([edge_index, jnp.tile(dummy, (2, 1))], axis=1)
    src_p = ei_p[0].reshape(NW * CH_PER_TILE, CHUNK)
    dst_p = ei_p[1].reshape(NW * CH_PER_TILE, CHUNK)
    x_p = jnp.pad(x, ((0, N_PAD - N), (0, 0)))
    batch_row = jnp.pad(batch, (0, N_PAD - N), constant_values=G).reshape(1, N_PAD)

    w1t = W1.T
    w2t = W2.T
    l1wt = l1w.T
    l2wt = l2w.T
    row = lambda v: v.reshape(1, -1)

    deg_parts = _sc_degree(dst_p)
    lin1 = _call(_tc_lin1, jax.ShapeDtypeStruct((N_PAD, H), f32),
                 x_p, w1t)
    u1, dinv = _call(_tc_scale,
                     (jax.ShapeDtypeStruct((N_PAD, H), f32),
                      jax.ShapeDtypeStruct((N_PAD, 1), f32)),
                     deg_parts, lin1)
    agg1 = _sc_aggregate(u1, src_p, dst_p)
    u2 = _call(_tc_mid, jax.ShapeDtypeStruct((N_PAD, H), f32),
               agg1[0], agg1[1], u1, dinv, row(b1), row(g1), row(be1),
               row(rm1), row(rv1), w2t)
    agg2 = _sc_aggregate(u2, src_p, dst_p)
    out = _call(_tc_head, jax.ShapeDtypeStruct((G, 1), f32),
                agg2[0], agg2[1], u2, dinv, row(b2), row(g2), row(be2),
                row(rm2), row(rv2), batch_row, l1wt, row(l1b), l2wt,
                l2b.reshape(1, 1))
    return out


# speedup vs baseline: 50.9036x; 1.2403x over previous
import functools

import jax
import jax.numpy as jnp
from jax import lax
from jax.experimental import pallas as pl
from jax.experimental.pallas import tpu as pltpu
from jax.experimental.pallas import tpu_sc as plsc

N = 10000
E = 320000
F_IN = 128
H = 64
G = 64
EPS = 1e-5

N_PAD = 10240
CHUNK = 128
NC, NS = 2, 16
NW = NC * NS
CH_PER_TILE = 80
E_PAD = NW * CH_PER_TILE * CHUNK
ROWS_PER_TILE = N_PAD // NS
KDEPTH = 8

_mesh = plsc.VectorSubcoreMesh(core_axis_name="c", subcore_axis_name="s")


DEGW = 16


@functools.partial(
    pl.kernel,
    out_type=jax.ShapeDtypeStruct((NC, N_PAD, DEGW), jnp.float32),
    mesh=_mesh,
    scratch_types=[
        pltpu.VMEM((CH_PER_TILE, CHUNK), jnp.int32),
        pltpu.VMEM((CHUNK, DEGW), jnp.float32),
        pltpu.VMEM((CHUNK, DEGW), jnp.float32),
        pltpu.VMEM_SHARED((N_PAD, DEGW), jnp.float32),
    ],
    compiler_params=pltpu.CompilerParams(use_tc_tiling_on_sc=False),
)
def _sc_degree(dst_hbm, out_hbm, idx_v, ones_v, zero_v, acc):
    c = lax.axis_index("c")
    s = lax.axis_index("s")
    w = c * NS + s
    pltpu.sync_copy(dst_hbm.at[pl.ds(w * CH_PER_TILE, CH_PER_TILE)], idx_v)

    def fill(i, carry):
        ones_v[i, :] = jnp.full((DEGW,), 1.0, jnp.float32)
        zero_v[i, :] = jnp.zeros((DEGW,), jnp.float32)
        return carry

    lax.fori_loop(0, CHUNK, fill, 0)
    for k in range(ROWS_PER_TILE // CHUNK):
        pltpu.sync_copy(zero_v,
                        acc.at[pl.ds(s * ROWS_PER_TILE + k * CHUNK, CHUNK)])
    plsc.subcore_barrier()

    def body(j, carry):
        pltpu.sync_copy(ones_v, acc.at[idx_v.at[j]], add=True)
        return carry

    lax.fori_loop(0, CH_PER_TILE, body, 0)
    plsc.subcore_barrier()
    pltpu.sync_copy(acc.at[pl.ds(s * ROWS_PER_TILE, ROWS_PER_TILE)],
                    out_hbm.at[c, pl.ds(s * ROWS_PER_TILE, ROWS_PER_TILE)])


@functools.partial(
    pl.kernel,
    out_type=jax.ShapeDtypeStruct((NC, N_PAD, H), jnp.float32),
    mesh=_mesh,
    scratch_types=[
        pltpu.VMEM((CH_PER_TILE + KDEPTH, CHUNK), jnp.int32),
        pltpu.VMEM((CH_PER_TILE, CHUNK), jnp.int32),
        [pltpu.VMEM((CHUNK, H), jnp.float32)] * KDEPTH,
        pltpu.VMEM_SHARED((N_PAD, H), jnp.float32),
        [pltpu.SemaphoreType.DMA] * KDEPTH,
        [pltpu.SemaphoreType.DMA] * KDEPTH,
    ],
    compiler_params=pltpu.CompilerParams(use_tc_tiling_on_sc=False),
)
def _sc_aggregate(u_hbm, src_hbm, dst_hbm, out_hbm,
                  src_v, dst_v, bufs, acc, gsems, ssems):
    c = lax.axis_index("c")
    s = lax.axis_index("s")
    w = c * NS + s
    pltpu.sync_copy(src_hbm.at[pl.ds(w * CH_PER_TILE, CH_PER_TILE)],
                    src_v.at[pl.ds(0, CH_PER_TILE)])
    pltpu.sync_copy(src_hbm.at[pl.ds(w * CH_PER_TILE, KDEPTH)],
                    src_v.at[pl.ds(CH_PER_TILE, KDEPTH)])
    pltpu.sync_copy(dst_hbm.at[pl.ds(w * CH_PER_TILE, CH_PER_TILE)], dst_v)

    def zfill(i, carry):
        for q in range(H // 16):
            bufs[0][i, pl.ds(q * 16, 16)] = jnp.zeros((16,), jnp.float32)
        return carry

    lax.fori_loop(0, CHUNK, zfill, 0)
    for k in range(ROWS_PER_TILE // CHUNK):
        pltpu.sync_copy(bufs[0],
                        acc.at[pl.ds(s * ROWS_PER_TILE + k * CHUNK, CHUNK)])
    gprime = [
        pltpu.async_copy(u_hbm.at[src_v.at[i]], bufs[i], gsems[i])
        for i in range(KDEPTH)
    ]
    plsc.subcore_barrier()

    def body(j, carry):
        base = j * KDEPTH
        scats = []
        for i in range(KDEPTH):
            pltpu.make_async_copy(u_hbm.at[src_v.at[base + i]], bufs[i],
                                  gsems[i]).wait()
            scats.append(pltpu.async_copy(
                bufs[i], acc.at[dst_v.at[base + i]], ssems[i], add=True))
        for i in range(KDEPTH):
            scats[i].wait()
            pltpu.async_copy(u_hbm.at[src_v.at[base + KDEPTH + i]], bufs[i],
                             gsems[i])
        return carry

    lax.fori_loop(0, CH_PER_TILE // KDEPTH, body, 0)
    for i in range(KDEPTH):
        pltpu.make_async_copy(u_hbm.at[src_v.at[CH_PER_TILE + i]], bufs[i],
                              gsems[i]).wait()
    plsc.subcore_barrier()
    pltpu.sync_copy(acc.at[pl.ds(s * ROWS_PER_TILE, ROWS_PER_TILE)],
                    out_hbm.at[c, pl.ds(s * ROWS_PER_TILE, ROWS_PER_TILE)])


FLAT = N_PAD * H // 128


def _tc_lin1(x_ref, w_ref, o_ref):
    o_ref[...] = jnp.dot(x_ref[...], w_ref[...],
                         preferred_element_type=jnp.float32)


def _tc_scale(degf_ref, lin_ref, u_ref):
    u_ref[...] = lin_ref[...] * lax.rsqrt(degf_ref[...] + 1.0)


def _tc_mid(aggf_ref, u_ref, degf_ref, b_ref, g_ref, be_ref, rm_ref,
            rv_ref, w2_ref, o_ref):
    dinv = lax.rsqrt(degf_ref[...] + 1.0)
    conv = dinv * (aggf_ref[0] + aggf_ref[1] + u_ref[...]) + b_ref[...]
    scale = g_ref[...] * lax.rsqrt(rv_ref[...] + EPS)
    h = jnp.maximum((conv - rm_ref[...]) * scale + be_ref[...], 0.0)
    o_ref[...] = jnp.dot(h, w2_ref[...],
                         preferred_element_type=jnp.float32) * dinv


def _tc_head(aggf_ref, u_ref, degf_ref, b_ref, g_ref, be_ref, rm_ref,
             rv_ref, be_ref2, bo_ref, l1w_ref, l1b_ref, l2w_ref, l2b_ref,
             o_ref):
    dinv = lax.rsqrt(degf_ref[...] + 1.0)
    conv = dinv * (aggf_ref[0] + aggf_ref[1] + u_ref[...]) + b_ref[...]
    scale = g_ref[...] * lax.rsqrt(rv_ref[...] + EPS)
    h = jnp.maximum((conv - rm_ref[...]) * scale + be_ref[...], 0.0)
    seg = lax.broadcasted_iota(jnp.int32, (G, FLAT), 0)
    ohe = (seg == be_ref2[...]).astype(jnp.float32)
    oho = (seg == bo_ref[...]).astype(jnp.float32)
    sums = (jnp.dot(ohe, h[:, 0:H], preferred_element_type=jnp.float32)
            + jnp.dot(oho, h[:, H:2 * H],
                      preferred_element_type=jnp.float32))
    ones = jnp.ones((FLAT, 1), jnp.float32)
    cnt = (jnp.dot(ohe, ones, preferred_element_type=jnp.float32)
           + jnp.dot(oho, ones, preferred_element_type=jnp.float32))
    pooled = sums / jnp.maximum(cnt, 1.0)
    z = jnp.maximum(jnp.dot(pooled, l1w_ref[...],
                            preferred_element_type=jnp.float32) + l1b_ref[...],
                    0.0)
    o_ref[...] = jnp.dot(z, l2w_ref[...],
                         preferred_element_type=jnp.float32) + l2b_ref[...]


def _call(body, out_shape, *args):
    return pl.pallas_call(body, out_shape=out_shape)(*args)


def kernel(x, edge_index, batch, W1, b1, W2, b2, g1, be1, rm1, rv1,
           g2, be2, rm2, rv2, l1w, l1b, l2w, l2b):
    f32 = jnp.float32
    src, dst = edge_index[0], edge_index[1]
    dummy = N + jnp.arange(E_PAD - E, dtype=jnp.int32) % (N_PAD - N)
    ei_p = jnp.concatenate([edge_index, jnp.tile(dummy, (2, 1))], axis=1)
    src_p = ei_p[0].reshape(NW * CH_PER_TILE, CHUNK)
    dst_p = ei_p[1].reshape(NW * CH_PER_TILE, CHUNK)
    x_p = jnp.pad(x, ((0, N_PAD - N), (0, 0)))
    batch_row = jnp.pad(batch, (0, N_PAD - N), constant_values=G).reshape(1, N_PAD)

    w1t = W1.T
    w2t = W2.T
    l1wt = l1w.T
    l2wt = l2w.T
    row = lambda v: v.reshape(1, -1)
    row2 = lambda v: jnp.concatenate([v, v]).reshape(1, 2 * H)
    zf = jnp.zeros((F_IN, H), f32)
    w1blk = jnp.concatenate(
        [jnp.concatenate([w1t, zf], axis=1),
         jnp.concatenate([zf, w1t], axis=1)], axis=0)
    zh = jnp.zeros((H, H), f32)
    w2blk = jnp.concatenate(
        [jnp.concatenate([w2t, zh], axis=1),
         jnp.concatenate([zh, w2t], axis=1)], axis=0)
    x_flat = x_p.reshape(FLAT, 2 * F_IN)
    batch_e = batch_row[0, 0::2].reshape(1, FLAT)
    batch_o = batch_row[0, 1::2].reshape(1, FLAT)

    deg_parts = _sc_degree(dst_p)
    lin1 = _call(_tc_lin1, jax.ShapeDtypeStruct((FLAT, 128), f32),
                 x_flat, w1blk)
    degsum = deg_parts[0, :, 0:1] + deg_parts[1, :, 0:1]
    deg_flat = jnp.broadcast_to(degsum, (N_PAD, H)).reshape(FLAT, 128)
    u1 = _call(_tc_scale, jax.ShapeDtypeStruct((FLAT, 128), f32),
               deg_flat, lin1)
    agg1 = _sc_aggregate(u1.reshape(N_PAD, H), src_p, dst_p)
    u2 = _call(_tc_mid, jax.ShapeDtypeStruct((FLAT, 128), f32),
               agg1.reshape(NC, FLAT, 128), u1, deg_flat, row2(b1),
               row2(g1), row2(be1), row2(rm1), row2(rv1), w2blk)
    agg2 = _sc_aggregate(u2.reshape(N_PAD, H), src_p, dst_p)
    out = _call(_tc_head, jax.ShapeDtypeStruct((G, 1), f32),
                agg2.reshape(NC, FLAT, 128), u2, deg_flat, row2(b2),
                row2(g2), row2(be2), row2(rm2), row2(rv2), batch_e, batch_o,
                l1wt, row(l1b), l2wt, l2b.reshape(1, 1))
    return out
